# baseline (speedup 1.0000x reference)
import functools

import jax
import jax.numpy as jnp
from jax import lax
from jax.experimental import pallas as pl
from jax.experimental.pallas import tpu as pltpu
from jax.experimental.pallas import tpu_sc as plsc

NW = 32
CHUNK = 128
D = 16


def _proj_body(xt_ref, w_ref, b_ref, o_ref):
    sub = o_ref.shape[0]
    for j in range(8):
        o = lax.dot_general(xt_ref[:, j * sub:(j + 1) * sub], w_ref[...],
                            (((0,), (0,)), ((), ())),
                            preferred_element_type=jnp.float32)
        o_ref[:, j * D:(j + 1) * D] = jnp.maximum(o + b_ref[...], 0.0)


def _tc_proj(features, W, b, block_cols):
    n, sd = features.shape
    nd = W.shape[1]
    xt = features.T
    grid = -(-n // block_cols)
    packed = pl.pallas_call(
        _proj_body,
        grid=(grid,),
        in_specs=[
            pl.BlockSpec((sd, block_cols), lambda i: (0, i)),
            pl.BlockSpec((sd, nd), lambda i: (0, 0)),
            pl.BlockSpec((1, nd), lambda i: (0, 0)),
        ],
        out_specs=pl.BlockSpec((block_cols * nd // 128, 128),
                               lambda i: (i, 0)),
        out_shape=jax.ShapeDtypeStruct(
            (grid * block_cols * nd // 128, 128), jnp.float32),
    )(xt, W, b.reshape(1, -1))
    return packed.reshape(grid * block_cols, nd)


def _permute_idx(idx, block_cols):
    t = idx % block_cols
    sub = block_cols // 8
    return (idx - t) + (t % sub) * 8 + t // sub


NBUF = 4


def _make_sc_scatter(K, MP):
    mesh = plsc.VectorSubcoreMesh(core_axis_name="c", subcore_axis_name="s")

    @functools.partial(
        pl.kernel,
        out_type=[
            jax.ShapeDtypeStruct((2, MP, D), jnp.float32),
            jax.ShapeDtypeStruct((2, MP, D), jnp.float32),
        ],
        mesh=mesh,
        scratch_types=[
            pltpu.VMEM((K, CHUNK), jnp.int32),
            pltpu.VMEM((K, CHUNK), jnp.int32),
            pltpu.VMEM((NBUF, CHUNK, D), jnp.float32),
            pltpu.VMEM((CHUNK, D), jnp.float32),
            pltpu.VMEM_SHARED((MP, D), jnp.float32),
            pltpu.VMEM_SHARED((MP, D), jnp.float32),
            pltpu.SemaphoreType.DMA((NBUF,)),
            pltpu.SemaphoreType.DMA,
            pltpu.SemaphoreType.DMA,
        ],
        compiler_params=pltpu.CompilerParams(use_tc_tiling_on_sc=False),
    )
    def sc_scatter(proj_hbm, idx_hbm, lab_hbm, ones_hbm, zeros_hbm,
                   zeros1_hbm, parts_hbm, cnts_hbm,
                   idx_v, lab_v, rows_v, ones_v, bank_sh, cnt_sh,
                   gsem, ssem, csem):
        c = lax.axis_index("c")
        s = lax.axis_index("s")
        wid = c * 16 + s

        @pl.when(s == 0)
        def _init():
            pltpu.sync_copy(zeros_hbm, bank_sh)
            pltpu.sync_copy(zeros_hbm, cnt_sh)

        pltpu.sync_copy(idx_hbm.at[wid], idx_v)
        pltpu.sync_copy(lab_hbm.at[wid], lab_v)
        pltpu.sync_copy(ones_hbm, ones_v)
        plsc.subcore_barrier()

        for k in range(min(NBUF, K)):
            pltpu.async_copy(proj_hbm.at[idx_v.at[k]], rows_v.at[k],
                             gsem.at[k])

        def body(j, carry):
            bsel = lax.rem(j, NBUF)
            pltpu.make_async_copy(proj_hbm.at[idx_v.at[j]], rows_v.at[bsel],
                                  gsem.at[bsel]).wait()
            s1 = pltpu.async_copy(rows_v.at[bsel],
                                  bank_sh.at[lab_v.at[j]], ssem, add=True)
            s2 = pltpu.async_copy(ones_v, cnt_sh.at[lab_v.at[j]], csem,
                                  add=True)
            s1.wait()
            s2.wait()
            nxt = j + NBUF

            @pl.when(nxt < K)
            def _prefetch():
                pltpu.async_copy(proj_hbm.at[idx_v.at[nxt]], rows_v.at[bsel],
                                 gsem.at[bsel])

            return carry

        lax.fori_loop(0, K, body, 0)
        plsc.subcore_barrier()

        @pl.when(s == 0)
        def _flush():
            pltpu.sync_copy(bank_sh, parts_hbm.at[c])
            pltpu.sync_copy(cnt_sh, cnts_hbm.at[c])

    return sc_scatter


def _ep_body(mem_ref, parts_ref, cnts_ref, o_ref):
    ssum = parts_ref[0] + parts_ref[1]
    cnt = cnts_ref[0, :, 0:1] + cnts_ref[1, :, 0:1]
    o_ref[...] = (mem_ref[...] + ssum) / (cnt + 1e-8)


def _tc_epilogue(memory, parts, cnts):
    M = memory.shape[0]
    return pl.pallas_call(
        _ep_body,
        in_specs=[
            pl.BlockSpec((M, D), lambda: (0, 0)),
            pl.BlockSpec((2, M, D), lambda: (0, 0, 0)),
            pl.BlockSpec((2, M, D), lambda: (0, 0, 0)),
        ],
        out_specs=pl.BlockSpec((M, D), lambda: (0, 0)),
        out_shape=jax.ShapeDtypeStruct((M, D), jnp.float32),
    )(memory, parts, cnts)


def kernel(features, idx, labels, memory, W, b):
    B = idx.shape[0]
    M = memory.shape[0]

    per_tile = -(-B // (NW * CHUNK)) * CHUNK
    K = per_tile // CHUNK
    b_pad = NW * per_tile - B
    mp = -(-(M + 1) // 8) * 8

    idx_p = jnp.concatenate(
        [_permute_idx(idx, 8192),
         jnp.zeros((b_pad,), jnp.int32)]).reshape(NW, K, CHUNK)
    lab_p = jnp.concatenate(
        [labels, jnp.full((b_pad,), M, jnp.int32)]).reshape(NW, K, CHUNK)
    ones_pat = jnp.zeros((CHUNK, D), jnp.float32).at[:, 0].set(1.0)
    zeros_pat = jnp.zeros((mp, D), jnp.float32)
    zeros1_pat = jnp.zeros((mp, 1), jnp.float32)

    proj = _tc_proj(features, W, b, block_cols=8192)
    parts, cnts = _make_sc_scatter(K, mp)(proj, idx_p, lab_p, ones_pat,
                                          zeros_pat, zeros1_pat)
    return _tc_epilogue(memory, parts[:, :M], cnts[:, :M])


# speedup vs baseline: 1.2851x; 1.2851x over previous
import functools

import jax
import jax.numpy as jnp
from jax import lax
from jax.experimental import pallas as pl
from jax.experimental.pallas import tpu as pltpu
from jax.experimental.pallas import tpu_sc as plsc

NW = 32
CHUNK = 128
D = 16


def _proj_body(xt_ref, w_ref, b_ref, o_ref):
    sub = o_ref.shape[0]
    o_all = lax.dot_general(xt_ref[...], w_ref[...], (((0,), (0,)), ((), ())),
                            preferred_element_type=jnp.float32)
    lane_grp = lax.broadcasted_iota(jnp.int32, (1, 8 * D), 1) // D
    out = o_all[0:sub, :]
    for j in range(1, 8):
        out = jnp.where(lane_grp == j, o_all[j * sub:(j + 1) * sub, :], out)
    o_ref[...] = jnp.maximum(out + b_ref[...], 0.0)


def _tc_proj(features, W, b, block_cols):
    n, sd = features.shape
    nd = W.shape[1]
    xt = features.T
    grid = -(-n // block_cols)
    packed = pl.pallas_call(
        _proj_body,
        grid=(grid,),
        in_specs=[
            pl.BlockSpec((sd, block_cols), lambda i: (0, i)),
            pl.BlockSpec((sd, 8 * nd), lambda i: (0, 0)),
            pl.BlockSpec((1, 8 * nd), lambda i: (0, 0)),
        ],
        out_specs=pl.BlockSpec((block_cols * nd // 128, 128),
                               lambda i: (i, 0)),
        out_shape=jax.ShapeDtypeStruct(
            (grid * block_cols * nd // 128, 128), jnp.float32),
        compiler_params=pltpu.CompilerParams(
            fuse_transposed_lhs_in_matmul=True),
    )(xt, jnp.tile(W, (1, 8)), jnp.tile(b, 8).reshape(1, -1))
    return packed.reshape(grid * block_cols, nd)


def _permute_idx(idx, block_cols):
    t = idx % block_cols
    sub = block_cols // 8
    return (idx - t) + (t % sub) * 8 + t // sub


NBUF = 4


def _make_sc_scatter(K, MP):
    mesh = plsc.VectorSubcoreMesh(core_axis_name="c", subcore_axis_name="s")

    @functools.partial(
        pl.kernel,
        out_type=[
            jax.ShapeDtypeStruct((2, MP, D), jnp.float32),
            jax.ShapeDtypeStruct((2, MP, D), jnp.float32),
        ],
        mesh=mesh,
        scratch_types=[
            pltpu.VMEM((K, CHUNK), jnp.int32),
            pltpu.VMEM((K, CHUNK), jnp.int32),
            pltpu.VMEM((NBUF, CHUNK, D), jnp.float32),
            pltpu.VMEM((CHUNK, D), jnp.float32),
            pltpu.VMEM_SHARED((MP, D), jnp.float32),
            pltpu.VMEM_SHARED((MP, D), jnp.float32),
            pltpu.SemaphoreType.DMA((NBUF,)),
            pltpu.SemaphoreType.DMA,
            pltpu.SemaphoreType.DMA,
        ],
        compiler_params=pltpu.CompilerParams(use_tc_tiling_on_sc=False),
    )
    def sc_scatter(proj_hbm, idx_hbm, lab_hbm, ones_hbm, zeros_hbm,
                   zeros1_hbm, parts_hbm, cnts_hbm,
                   idx_v, lab_v, rows_v, ones_v, bank_sh, cnt_sh,
                   gsem, ssem, csem):
        c = lax.axis_index("c")
        s = lax.axis_index("s")
        wid = c * 16 + s

        @pl.when(s == 0)
        def _init():
            pltpu.sync_copy(zeros_hbm, bank_sh)
            pltpu.sync_copy(zeros_hbm, cnt_sh)

        pltpu.sync_copy(idx_hbm.at[wid], idx_v)
        pltpu.sync_copy(lab_hbm.at[wid], lab_v)
        pltpu.sync_copy(ones_hbm, ones_v)
        plsc.subcore_barrier()

        for k in range(min(NBUF, K)):
            pltpu.async_copy(proj_hbm.at[idx_v.at[k]], rows_v.at[k],
                             gsem.at[k])

        def body(j, carry):
            bsel = lax.rem(j, NBUF)
            pltpu.make_async_copy(proj_hbm.at[idx_v.at[j]], rows_v.at[bsel],
                                  gsem.at[bsel]).wait()
            s1 = pltpu.async_copy(rows_v.at[bsel],
                                  bank_sh.at[lab_v.at[j]], ssem, add=True)
            s2 = pltpu.async_copy(ones_v, cnt_sh.at[lab_v.at[j]], csem,
                                  add=True)
            s1.wait()
            s2.wait()
            nxt = j + NBUF

            @pl.when(nxt < K)
            def _prefetch():
                pltpu.async_copy(proj_hbm.at[idx_v.at[nxt]], rows_v.at[bsel],
                                 gsem.at[bsel])

            return carry

        lax.fori_loop(0, K, body, 0)
        plsc.subcore_barrier()

        @pl.when(s == 0)
        def _flush():
            pltpu.sync_copy(bank_sh, parts_hbm.at[c])
            pltpu.sync_copy(cnt_sh, cnts_hbm.at[c])

    return sc_scatter


def _ep_body(mem_ref, parts_ref, cnts_ref, o_ref):
    ssum = parts_ref[0] + parts_ref[1]
    cnt = cnts_ref[0, :, 0:1] + cnts_ref[1, :, 0:1]
    o_ref[...] = (mem_ref[...] + ssum) / (cnt + 1e-8)


def _tc_epilogue(memory, parts, cnts):
    M = memory.shape[0]
    return pl.pallas_call(
        _ep_body,
        in_specs=[
            pl.BlockSpec((M, D), lambda: (0, 0)),
            pl.BlockSpec((2, M, D), lambda: (0, 0, 0)),
            pl.BlockSpec((2, M, D), lambda: (0, 0, 0)),
        ],
        out_specs=pl.BlockSpec((M, D), lambda: (0, 0)),
        out_shape=jax.ShapeDtypeStruct((M, D), jnp.float32),
    )(memory, parts, cnts)


def kernel(features, idx, labels, memory, W, b):
    B = idx.shape[0]
    M = memory.shape[0]

    per_tile = -(-B // (NW * CHUNK)) * CHUNK
    K = per_tile // CHUNK
    b_pad = NW * per_tile - B
    mp = -(-(M + 1) // 8) * 8

    idx_p = jnp.concatenate(
        [_permute_idx(idx, 8192),
         jnp.zeros((b_pad,), jnp.int32)]).reshape(NW, K, CHUNK)
    lab_p = jnp.concatenate(
        [labels, jnp.full((b_pad,), M, jnp.int32)]).reshape(NW, K, CHUNK)
    ones_pat = jnp.zeros((CHUNK, D), jnp.float32).at[:, 0].set(1.0)
    zeros_pat = jnp.zeros((mp, D), jnp.float32)
    zeros1_pat = jnp.zeros((mp, 1), jnp.float32)

    proj = _tc_proj(features, W, b, block_cols=8192)
    parts, cnts = _make_sc_scatter(K, mp)(proj, idx_p, lab_p, ones_pat,
                                          zeros_pat, zeros1_pat)
    return _tc_epilogue(memory, parts[:, :M], cnts[:, :M])


# speedup vs baseline: 1.4625x; 1.1380x over previous
import functools

import jax
import jax.numpy as jnp
from jax import lax
from jax.experimental import pallas as pl
from jax.experimental.pallas import tpu as pltpu
from jax.experimental.pallas import tpu_sc as plsc

NW = 32
CHUNK = 128
D = 16


def _proj_body(xt_ref, w_ref, b_ref, o_ref):
    sub = o_ref.shape[0]
    o_all = lax.dot_general(xt_ref[...], w_ref[...], (((0,), (0,)), ((), ())),
                            preferred_element_type=jnp.float32)
    lane_grp = lax.broadcasted_iota(jnp.int32, (1, 8 * D), 1) // D
    out = o_all[0:sub, :]
    for j in range(1, 8):
        out = jnp.where(lane_grp == j, o_all[j * sub:(j + 1) * sub, :], out)
    o_ref[...] = jnp.maximum(out + b_ref[...], 0.0)


def _tc_proj(features, W, b, block_cols):
    n, sd = features.shape
    nd = W.shape[1]
    xt = features.T
    grid = -(-n // block_cols)
    packed = pl.pallas_call(
        _proj_body,
        grid=(grid,),
        in_specs=[
            pl.BlockSpec((sd, block_cols), lambda i: (0, i)),
            pl.BlockSpec((sd, 8 * nd), lambda i: (0, 0)),
            pl.BlockSpec((1, 8 * nd), lambda i: (0, 0)),
        ],
        out_specs=pl.BlockSpec((block_cols * nd // 128, 128),
                               lambda i: (i, 0)),
        out_shape=jax.ShapeDtypeStruct(
            (grid * block_cols * nd // 128, 128), jnp.float32),
        compiler_params=pltpu.CompilerParams(
            fuse_transposed_lhs_in_matmul=True),
    )(xt, jnp.tile(W, (1, 8)), jnp.tile(b, 8).reshape(1, -1))
    return packed.reshape(grid * block_cols, nd)


def _permute_idx(idx, block_cols):
    t = idx % block_cols
    sub = block_cols // 8
    return (idx - t) + (t % sub) * 8 + t // sub


NBUF = 4


def _make_sc_scatter(K, MP):
    mesh = plsc.VectorSubcoreMesh(core_axis_name="c", subcore_axis_name="s")

    @functools.partial(
        pl.kernel,
        out_type=[
            jax.ShapeDtypeStruct((2, MP, D), jnp.float32),
            jax.ShapeDtypeStruct((2, MP, D), jnp.float32),
        ],
        mesh=mesh,
        scratch_types=[
            pltpu.VMEM((K, CHUNK), jnp.int32),
            pltpu.VMEM((K, CHUNK), jnp.int32),
            pltpu.VMEM((NBUF, CHUNK, D), jnp.float32),
            pltpu.VMEM((CHUNK, D), jnp.float32),
            pltpu.VMEM_SHARED((MP, D), jnp.float32),
            pltpu.VMEM_SHARED((MP, D), jnp.float32),
            pltpu.SemaphoreType.DMA((NBUF,)),
            pltpu.SemaphoreType.DMA,
            pltpu.SemaphoreType.DMA,
        ],
        compiler_params=pltpu.CompilerParams(use_tc_tiling_on_sc=False),
    )
    def sc_scatter(proj_hbm, idx_hbm, lab_hbm, ones_hbm, zeros_hbm,
                   zeros1_hbm, parts_hbm, cnts_hbm,
                   idx_v, lab_v, rows_v, ones_v, bank_sh, cnt_sh,
                   gsem, ssem, csem):
        c = lax.axis_index("c")
        s = lax.axis_index("s")
        wid = c * 16 + s

        @pl.when(s == 0)
        def _init():
            pltpu.sync_copy(zeros_hbm, bank_sh)
            pltpu.sync_copy(zeros_hbm, cnt_sh)

        pltpu.sync_copy(idx_hbm.at[wid], idx_v)
        pltpu.sync_copy(lab_hbm.at[wid], lab_v)
        pltpu.sync_copy(ones_hbm, ones_v)
        plsc.subcore_barrier()

        for k in range(min(NBUF, K)):
            pltpu.async_copy(proj_hbm.at[idx_v.at[k]], rows_v.at[k],
                             gsem.at[k])

        def body(j, carry):
            bsel = lax.rem(j, NBUF)
            pltpu.make_async_copy(proj_hbm.at[idx_v.at[j]], rows_v.at[bsel],
                                  gsem.at[bsel]).wait()
            s1 = pltpu.async_copy(rows_v.at[bsel],
                                  bank_sh.at[lab_v.at[j]], ssem, add=True)
            s2 = pltpu.async_copy(ones_v, cnt_sh.at[lab_v.at[j]], csem,
                                  add=True)
            s1.wait()
            s2.wait()
            nxt = j + NBUF

            @pl.when(nxt < K)
            def _prefetch():
                pltpu.async_copy(proj_hbm.at[idx_v.at[nxt]], rows_v.at[bsel],
                                 gsem.at[bsel])

            return carry

        lax.fori_loop(0, K, body, 0)
        plsc.subcore_barrier()

        @pl.when(s == 0)
        def _flush():
            pltpu.sync_copy(bank_sh, parts_hbm.at[c])
            pltpu.sync_copy(cnt_sh, cnts_hbm.at[c])

    return sc_scatter


def _ep_body(mem_ref, parts_ref, cnts_ref, o_ref):
    ssum = parts_ref[0] + parts_ref[1]
    cnt = cnts_ref[0, :, 0:1] + cnts_ref[1, :, 0:1]
    o_ref[...] = (mem_ref[...] + ssum) / (cnt + 1e-8)


def _tc_epilogue(memory, parts, cnts):
    M = memory.shape[0]
    return pl.pallas_call(
        _ep_body,
        in_specs=[
            pl.BlockSpec((M, D), lambda: (0, 0)),
            pl.BlockSpec((2, M, D), lambda: (0, 0, 0)),
            pl.BlockSpec((2, M, D), lambda: (0, 0, 0)),
        ],
        out_specs=pl.BlockSpec((M, D), lambda: (0, 0)),
        out_shape=jax.ShapeDtypeStruct((M, D), jnp.float32),
    )(memory, parts, cnts)


def kernel(features, idx, labels, memory, W, b):
    B = idx.shape[0]
    M = memory.shape[0]

    per_tile = -(-B // (NW * CHUNK)) * CHUNK
    K = per_tile // CHUNK
    b_pad = NW * per_tile - B
    mp = -(-(M + 1) // 8) * 8

    idx_p = jnp.concatenate(
        [_permute_idx(idx, 16384),
         jnp.zeros((b_pad,), jnp.int32)]).reshape(NW, K, CHUNK)
    lab_p = jnp.concatenate(
        [labels, jnp.full((b_pad,), M, jnp.int32)]).reshape(NW, K, CHUNK)
    ones_pat = jnp.zeros((CHUNK, D), jnp.float32).at[:, 0].set(1.0)
    zeros_pat = jnp.zeros((mp, D), jnp.float32)
    zeros1_pat = jnp.zeros((mp, 1), jnp.float32)

    proj = _tc_proj(features, W, b, block_cols=16384)
    parts, cnts = _make_sc_scatter(K, mp)(proj, idx_p, lab_p, ones_pat,
                                          zeros_pat, zeros1_pat)
    return _tc_epilogue(memory, parts[:, :M], cnts[:, :M])
